# baseline (speedup 1.0000x reference)
import functools

import jax
import jax.numpy as jnp
from jax import lax
from jax.experimental import pallas as pl
from jax.experimental.pallas import tpu as pltpu
from jax.experimental.pallas import tpu_sc as plsc

_NUM_CORES = 2
_NUM_SUBCORES = 16


@functools.lru_cache(maxsize=None)
def _make_copy_kernel(t, d, dtype):
    num_workers = _NUM_CORES * _NUM_SUBCORES
    rows_per_w = t // num_workers
    assert rows_per_w * num_workers == t

    mesh = plsc.VectorSubcoreMesh(core_axis_name="c", subcore_axis_name="s")

    @functools.partial(
        pl.kernel,
        mesh=mesh,
        out_type=jax.ShapeDtypeStruct((t, d), dtype),
    )
    def copy_rows(table_hbm, out_hbm):
        wid = lax.axis_index("s") * _NUM_CORES + lax.axis_index("c")
        base = wid * rows_per_w
        pltpu.sync_copy(
            table_hbm.at[pl.ds(base, rows_per_w)],
            out_hbm.at[pl.ds(base, rows_per_w)],
        )

    return copy_rows


def kernel(input_ids, emb_weight):
    b, t = input_ids.shape
    d = emb_weight.shape[1]
    out = _make_copy_kernel(t, d, emb_weight.dtype)(emb_weight)
    return out[None]
